# initial kernel scaffold (unmeasured)
import jax
import jax.numpy as jnp
from jax import lax
from jax.experimental import pallas as pl
from jax.experimental.pallas import tpu as pltpu


def kernel(
    x,
):
    def body(*refs):
        pass

    out_shape = jax.ShapeDtypeStruct(..., jnp.float32)
    return pl.pallas_call(body, out_shape=out_shape)(...)


# baseline (device time: 22888 ns/iter reference)
import jax
import jax.numpy as jnp
from jax import lax
from jax.experimental import pallas as pl
from jax.experimental.pallas import tpu as pltpu

N_DEV = 8
M = 256
N_CHUNK = 256


def kernel(x):
    _, m, n_tot = x.shape
    assert (m, n_tot) == (M, N_DEV * N_CHUNK), x.shape

    def body(x_ref, out_ref, comm_ref, send_sems, recv_sems, local_sem):
        my_pos = lax.axis_index("i")

        barrier_sem = pltpu.get_barrier_semaphore()
        for k in range(1, N_DEV):
            peer = lax.rem(my_pos + k, N_DEV)
            pl.semaphore_signal(
                barrier_sem, inc=1,
                device_id=(peer,), device_id_type=pl.DeviceIdType.MESH,
            )
        pl.semaphore_wait(barrier_sem, N_DEV - 1)

        own = pltpu.make_async_copy(
            x_ref.at[0, :, pl.ds(my_pos * N_CHUNK, N_CHUNK)],
            comm_ref.at[my_pos],
            local_sem,
        )
        own.start()

        sends = []
        for k in range(1, N_DEV):
            dst = lax.rem(my_pos + k, N_DEV)
            rdma = pltpu.make_async_remote_copy(
                src_ref=x_ref.at[0, :, pl.ds(dst * N_CHUNK, N_CHUNK)],
                dst_ref=comm_ref.at[my_pos],
                send_sem=send_sems.at[k],
                recv_sem=recv_sems.at[my_pos],
                device_id=(dst,),
                device_id_type=pl.DeviceIdType.MESH,
            )
            rdma.start()
            sends.append(rdma)

        own.wait()

        for k in range(1, N_DEV):
            src = lax.rem(my_pos + k, N_DEV)
            recv = pltpu.make_async_remote_copy(
                src_ref=comm_ref.at[src],
                dst_ref=comm_ref.at[src],
                send_sem=send_sems.at[k],
                recv_sem=recv_sems.at[src],
                device_id=(src,),
                device_id_type=pl.DeviceIdType.MESH,
            )
            recv.wait_recv()

        acc = comm_ref[0]
        for s in range(1, N_DEV):
            acc = acc + comm_ref[s]
        out_ref[:, :] = acc

        for rdma in sends:
            rdma.wait_send()

    return pl.pallas_call(
        body,
        out_shape=jax.ShapeDtypeStruct((M, N_CHUNK), x.dtype),
        in_specs=[pl.BlockSpec(memory_space=pltpu.VMEM)],
        out_specs=pl.BlockSpec(memory_space=pltpu.VMEM),
        scratch_shapes=[
            pltpu.VMEM((N_DEV, M, N_CHUNK), x.dtype),
            pltpu.SemaphoreType.DMA((N_DEV,)),
            pltpu.SemaphoreType.DMA((N_DEV,)),
            pltpu.SemaphoreType.DMA,
        ],
        compiler_params=pltpu.CompilerParams(collective_id=0),
    )(x)


# device time: 19379 ns/iter; 1.1811x vs baseline; 1.1811x over previous
import jax
import jax.numpy as jnp
from jax import lax
from jax.experimental import pallas as pl
from jax.experimental.pallas import tpu as pltpu

N_DEV = 8
M = 256
N_CHUNK = 256

ORDERS = ((1, 3, 4), (3, 4, 1), (4, 1, 3))
ROW_START = (0, 96, 192)
ROW_LEN = (96, 96, 64)
SLOT_BASE = (0, 4, 6)


def _span(gens):
    s = {0}
    for g in gens:
        s |= {e ^ g for e in s}
    return sorted(s)


def kernel(x):
    _, m, n_tot = x.shape
    assert (m, n_tot) == (M, N_DEV * N_CHUNK), x.shape

    def body(x_ref, out_ref, acc_ref, recv_ref, send_sems, recv_sems, init_sems):
        my_pos = lax.axis_index("i")

        inits = []
        for r in range(N_DEV):
            c = jnp.bitwise_xor(my_pos, r)
            cp = pltpu.make_async_copy(
                x_ref.at[0, :, pl.ds(c * N_CHUNK, N_CHUNK)],
                acc_ref.at[r],
                init_sems.at[r],
            )
            cp.start()
            inits.append(cp)

        barrier_sem = pltpu.get_barrier_semaphore()
        for g in (1, 3, 4):
            pl.semaphore_signal(
                barrier_sem, inc=1,
                device_id=(jnp.bitwise_xor(my_pos, g),),
                device_id_type=pl.DeviceIdType.MESH,
            )
        pl.semaphore_wait(barrier_sem, 3)

        all_sends = []
        for step in range(3):
            recvs = []
            for p in range(3):
                g0, g1, g2 = ORDERS[p]
                gs = ORDERS[p][step]
                rest = ORDERS[p][step + 1:]
                es = _span(rest)
                dst = jnp.bitwise_xor(my_pos, gs)
                rs, rl = ROW_START[p], ROW_LEN[p]
                for j, e in enumerate(es):
                    slot = SLOT_BASE[step] + j
                    if step == 0:
                        c = jnp.bitwise_xor(dst, e)
                        src = x_ref.at[0, pl.ds(rs, rl), pl.ds(c * N_CHUNK, N_CHUNK)]
                    else:
                        src = acc_ref.at[gs ^ e, pl.ds(rs, rl), :]
                    rdma = pltpu.make_async_remote_copy(
                        src_ref=src,
                        dst_ref=recv_ref.at[p, slot, pl.ds(0, rl), :],
                        send_sem=send_sems.at[step, p, j],
                        recv_sem=recv_sems.at[step, p, j],
                        device_id=(dst,),
                        device_id_type=pl.DeviceIdType.MESH,
                    )
                    rdma.start()
                    all_sends.append(rdma)
                    recvs.append((p, slot, j, es, rs, rl))

            if step == 0:
                for cp in inits:
                    cp.wait()

            for p in range(3):
                rest = ORDERS[p][step + 1:]
                es = _span(rest)
                rl = ROW_LEN[p]
                for j, e in enumerate(es):
                    slot = SLOT_BASE[step] + j
                    recv = pltpu.make_async_remote_copy(
                        src_ref=recv_ref.at[p, slot, pl.ds(0, rl), :],
                        dst_ref=recv_ref.at[p, slot, pl.ds(0, rl), :],
                        send_sem=send_sems.at[step, p, j],
                        recv_sem=recv_sems.at[step, p, j],
                        device_id=(jnp.bitwise_xor(my_pos, ORDERS[p][step]),),
                        device_id_type=pl.DeviceIdType.MESH,
                    )
                    recv.wait_recv()
            for p in range(3):
                rest = ORDERS[p][step + 1:]
                es = _span(rest)
                rs, rl = ROW_START[p], ROW_LEN[p]
                for j, e in enumerate(es):
                    slot = SLOT_BASE[step] + j
                    acc_ref[e, pl.ds(rs, rl), :] = (
                        acc_ref[e, pl.ds(rs, rl), :]
                        + recv_ref[p, slot, pl.ds(0, rl), :]
                    )

        out_ref[:, :] = acc_ref[0]

        for rdma in all_sends:
            rdma.wait_send()

    return pl.pallas_call(
        body,
        out_shape=jax.ShapeDtypeStruct((M, N_CHUNK), x.dtype),
        in_specs=[pl.BlockSpec(memory_space=pltpu.VMEM)],
        out_specs=pl.BlockSpec(memory_space=pltpu.VMEM),
        scratch_shapes=[
            pltpu.VMEM((N_DEV, M, N_CHUNK), x.dtype),
            pltpu.VMEM((3, 7, max(ROW_LEN), N_CHUNK), x.dtype),
            pltpu.SemaphoreType.DMA((3, 3, 4)),
            pltpu.SemaphoreType.DMA((3, 3, 4)),
            pltpu.SemaphoreType.DMA((N_DEV,)),
        ],
        compiler_params=pltpu.CompilerParams(collective_id=0),
    )(x)


# device time: 19211 ns/iter; 1.1914x vs baseline; 1.0087x over previous
import jax
import jax.numpy as jnp
from jax import lax
from jax.experimental import pallas as pl
from jax.experimental.pallas import tpu as pltpu

N_DEV = 8
M = 256
N_CHUNK = 256

ORDERS = ((1, 3, 4), (3, 4, 1), (4, 1, 3))
ROW_START = (0, 96, 192)
ROW_LEN = (96, 96, 64)
SLOT_BASE = (0, 4, 6)


def _span(gens):
    s = {0}
    for g in gens:
        s |= {e ^ g for e in s}
    return sorted(s)


def kernel(x):
    _, m, n_tot = x.shape
    assert (m, n_tot) == (M, N_DEV * N_CHUNK), x.shape

    def body(x_ref, out_ref, acc_ref, recv_ref, send_sems, recv_sems, init_sems):
        my_pos = lax.axis_index("i")

        inits = []
        for r in range(N_DEV):
            c = jnp.bitwise_xor(my_pos, r)
            cp = pltpu.make_async_copy(
                x_ref.at[0, :, pl.ds(c * N_CHUNK, N_CHUNK)],
                acc_ref.at[r],
                init_sems.at[r],
            )
            cp.start()
            inits.append(cp)

        barrier_sem = pltpu.get_barrier_semaphore()
        for g in (1, 3, 4):
            pl.semaphore_signal(
                barrier_sem, inc=1,
                device_id=(jnp.bitwise_xor(my_pos, g),),
                device_id_type=pl.DeviceIdType.MESH,
            )
        pl.semaphore_wait(barrier_sem, 3)

        all_sends = []

        def issue(step, p):
            gs = ORDERS[p][step]
            es = _span(ORDERS[p][step + 1:])
            dst = jnp.bitwise_xor(my_pos, gs)
            rs, rl = ROW_START[p], ROW_LEN[p]
            for j, e in enumerate(es):
                slot = SLOT_BASE[step] + j
                if step == 0:
                    c = jnp.bitwise_xor(dst, e)
                    src = x_ref.at[0, pl.ds(rs, rl), pl.ds(c * N_CHUNK, N_CHUNK)]
                else:
                    src = acc_ref.at[gs ^ e, pl.ds(rs, rl), :]
                rdma = pltpu.make_async_remote_copy(
                    src_ref=src,
                    dst_ref=recv_ref.at[p, slot, pl.ds(0, rl), :],
                    send_sem=send_sems.at[step, p, j],
                    recv_sem=recv_sems.at[step, p, j],
                    device_id=(dst,),
                    device_id_type=pl.DeviceIdType.MESH,
                )
                rdma.start()
                all_sends.append(rdma)

        def wait_and_reduce(step, p):
            es = _span(ORDERS[p][step + 1:])
            rs, rl = ROW_START[p], ROW_LEN[p]
            for j, e in enumerate(es):
                slot = SLOT_BASE[step] + j
                recv = pltpu.make_async_remote_copy(
                    src_ref=recv_ref.at[p, slot, pl.ds(0, rl), :],
                    dst_ref=recv_ref.at[p, slot, pl.ds(0, rl), :],
                    send_sem=send_sems.at[step, p, j],
                    recv_sem=recv_sems.at[step, p, j],
                    device_id=(jnp.bitwise_xor(my_pos, ORDERS[p][step]),),
                    device_id_type=pl.DeviceIdType.MESH,
                )
                recv.wait_recv()
                acc_ref[e, pl.ds(rs, rl), :] = (
                    acc_ref[e, pl.ds(rs, rl), :]
                    + recv_ref[p, slot, pl.ds(0, rl), :]
                )

        for p in range(3):
            issue(0, p)
        for cp in inits:
            cp.wait()
        for p in range(3):
            wait_and_reduce(0, p)
            issue(1, p)
        for p in range(3):
            wait_and_reduce(1, p)
            issue(2, p)
        for p in range(3):
            wait_and_reduce(2, p)

        out_ref[:, :] = acc_ref[0]

        for rdma in all_sends:
            rdma.wait_send()

    return pl.pallas_call(
        body,
        out_shape=jax.ShapeDtypeStruct((M, N_CHUNK), x.dtype),
        in_specs=[pl.BlockSpec(memory_space=pltpu.VMEM)],
        out_specs=pl.BlockSpec(memory_space=pltpu.VMEM),
        scratch_shapes=[
            pltpu.VMEM((N_DEV, M, N_CHUNK), x.dtype),
            pltpu.VMEM((3, 7, max(ROW_LEN), N_CHUNK), x.dtype),
            pltpu.SemaphoreType.DMA((3, 3, 4)),
            pltpu.SemaphoreType.DMA((3, 3, 4)),
            pltpu.SemaphoreType.DMA((N_DEV,)),
        ],
        compiler_params=pltpu.CompilerParams(collective_id=0),
    )(x)
